# baseline (speedup 1.0000x reference)
import jax
import jax.numpy as jnp
from jax.experimental import pallas as pl
from jax.experimental.pallas import tpu as pltpu


def _body(trow_ref, ts_ref, es_ref, o_ref, w_s, a_s, d_s, x_s, m_s):
    i = pl.program_id(0)
    m = ts_ref.shape[1]
    B = trow_ref.shape[1]
    k = es_ref.shape[1]
    BR = o_ref.shape[0]
    CHI = BR // k

    @pl.when(i == 0)
    def _prep():
        ts = ts_ref[:, :]
        lo = ts[0, 0]
        hi = ts[0, m - 1]
        tr = trow_ref[:, :]
        trc = jnp.clip(tr, lo, hi)
        idxc = jnp.zeros(tr.shape, jnp.int32)
        for mm in range(m):
            idxc += (ts[0, mm] < trc).astype(jnp.int32)
        idxc = jnp.clip(idxc, 1, m - 1)
        t0 = jnp.zeros(tr.shape, jnp.float32)
        t1 = jnp.zeros(tr.shape, jnp.float32)
        for mm in range(m):
            t0 = jnp.where(idxc - 1 == mm, ts[0, mm], t0)
            t1 = jnp.where(idxc == mm, ts[0, mm], t1)
        w_s[:, :] = (trc - t0) / (t1 - t0 + 1e-12)

        rows = jax.lax.broadcasted_iota(jnp.int32, (m, B), 0)
        p0 = (rows == (idxc - 1)).astype(jnp.float32)
        p1 = (rows == idxc).astype(jnp.float32)
        es = es_ref[:, :]
        dn = (((0,), (0,)), ((), ()))
        e0 = jax.lax.dot_general(p0, es, dn,
                                 preferred_element_type=jnp.float32)
        e1 = jax.lax.dot_general(p1, es, dn,
                                 preferred_element_type=jnp.float32)
        a_s[:, :] = e0
        d_s[:, :] = e1 - e0

        rr = jax.lax.broadcasted_iota(jnp.int32, (BR, CHI), 0)
        qq = jax.lax.broadcasted_iota(jnp.int32, (BR, CHI), 1)
        x_s[:, :] = (qq == rr // k).astype(jnp.float32)
        r2 = jax.lax.broadcasted_iota(jnp.int32, (BR, k), 0)
        ss = jax.lax.broadcasted_iota(jnp.int32, (BR, k), 1)
        m_s[:, :] = (ss == r2 % k).astype(jnp.float32)

    x = x_s[:, :]
    msk = m_s[:, :]
    a_blk = a_s[pl.ds(i * CHI, CHI), :]
    d_blk = d_s[pl.ds(i * CHI, CHI), :]
    ua = jnp.dot(x, a_blk, preferred_element_type=jnp.float32)
    ud = jnp.dot(x, d_blk, preferred_element_type=jnp.float32)
    a_col = jnp.sum(ua * msk, axis=1, keepdims=True)
    d_col = jnp.sum(ud * msk, axis=1, keepdims=True)
    o_ref[:, :] = a_col + d_col * w_s[:, :]


def kernel(t, ts, Es):
    B = t.shape[0]
    m = ts.shape[0]
    k = Es.shape[1]
    R = B * k

    ts2 = ts.reshape(1, m)
    trow = t.reshape(1, B)

    BR = 512
    CHI = BR // k
    q = pl.pallas_call(
        _body,
        grid=(R // BR,),
        in_specs=[
            pl.BlockSpec((1, B), lambda i: (0, 0)),
            pl.BlockSpec((1, m), lambda i: (0, 0)),
            pl.BlockSpec((m, k), lambda i: (0, 0)),
        ],
        out_specs=pl.BlockSpec((BR, B), lambda i: (i, 0)),
        out_shape=jax.ShapeDtypeStruct((R, B), jnp.float32),
        scratch_shapes=[
            pltpu.VMEM((1, B), jnp.float32),
            pltpu.VMEM((B, k), jnp.float32),
            pltpu.VMEM((B, k), jnp.float32),
            pltpu.VMEM((BR, CHI), jnp.float32),
            pltpu.VMEM((BR, k), jnp.float32),
        ],
    )(trow, ts2, Es)

    return q.reshape(B, k, B).transpose(0, 2, 1)
